# baseline (speedup 1.0000x reference)
import functools

import jax
import jax.numpy as jnp
from jax import lax
from jax.experimental import pallas as pl
from jax.experimental.pallas import tpu as pltpu
from jax.experimental.pallas import tpu_sc as plsc

_NC = 2
_NS = 16
_L = 16
_NW = _NC * _NS


def _knn_body(x_ref, w1_ref, idx_ref, w1e_ref, *, n, k, f):
    b = pl.program_id(0)
    xf = x_ref[0]
    sq = jnp.sum(xf * xf, axis=1)
    g = lax.dot_general(xf, xf, (((1,), (1,)), ((), ())),
                        preferred_element_type=jnp.float32)
    dist = sq[:, None] - 2.0 * g + sq[None, :]
    iota = lax.broadcasted_iota(jnp.int32, (n, n), 1)
    ams = []
    for _ in range(k):
        am = jnp.argmin(dist, axis=1).astype(jnp.int32)
        ams.append(am)
        dist = jnp.where(iota == am[:, None], jnp.inf, dist)
    idx = jnp.stack(ams, axis=1)
    idx_ref[0] = idx + b * n

    @pl.when(b == 0)
    def _():
        w1n = w1_ref[...] ** 2
        w1n = w1n / jnp.sum(w1n, axis=1, keepdims=True)
        w1t = w1n.T
        w1e_ref[...] = jnp.concatenate([w1t] * (f // w1t.shape[1]), axis=1)


def _sc_gather_combine(table, idxf, w1e, *, k, f, rows_per_w):
    nrows = table.shape[0]
    fc = w1e.shape[1]
    mesh = plsc.VectorSubcoreMesh(core_axis_name="c", subcore_axis_name="s")

    pt = 4
    ng = rows_per_w // pt

    @functools.partial(
        pl.kernel,
        out_type=jax.ShapeDtypeStruct((nrows, f), jnp.float32),
        mesh=mesh,
        scratch_types=[
            pltpu.VMEM((rows_per_w * k,), jnp.int32),
            pltpu.VMEM((2, pt * k, f), jnp.float32),
            pltpu.VMEM((rows_per_w, f), jnp.float32),
            pltpu.VMEM((k, fc), jnp.float32),
            pltpu.SemaphoreType.DMA,
            pltpu.SemaphoreType.DMA,
        ],
    )
    def kb(table_hbm, idx_hbm, w1e_hbm, out_hbm, idx_v, rows_v, out_v,
           w1e_v, sem0, sem1):
        wid = lax.axis_index("s") * _NC + lax.axis_index("c")
        base = wid * rows_per_w
        pltpu.sync_copy(idx_hbm.at[pl.ds(base * k, rows_per_w * k)], idx_v)
        pltpu.sync_copy(w1e_hbm, w1e_v)
        sems = (sem0, sem1)

        def gather(slot, g):
            pltpu.make_async_copy(
                table_hbm.at[idx_v.at[pl.ds(g * pt * k, pt * k)]],
                rows_v.at[slot], sems[slot],
            ).start()

        def wait(slot):
            pltpu.make_async_copy(
                table_hbm.at[idx_v.at[pl.ds(0, pt * k)]],
                rows_v.at[slot], sems[slot],
            ).wait()

        def combine(slot, g):
            for c in range(fc // _L):
                sl = pl.ds(c * _L, _L)
                acc = [None] * pt
                for j in range(k):
                    wv = w1e_v[j, sl]
                    for p in range(pt):
                        term = rows_v[slot, p * k + j, sl] * wv
                        acc[p] = term if j == 0 else acc[p] + term
                for p in range(pt):
                    out_v[g * pt + p, sl] = acc[p]

        gather(0, 0)

        def body(i, _):
            g0 = 2 * i
            gather(1, g0 + 1)
            wait(0)
            combine(0, g0)

            @pl.when(i + 1 < ng // 2)
            def _():
                gather(0, g0 + 2)

            wait(1)
            combine(1, g0 + 1)
            return 0

        lax.fori_loop(0, ng // 2, body, 0)
        pltpu.sync_copy(out_v, out_hbm.at[pl.ds(base, rows_per_w)])

    return kb(table, idxf, w1e)


def _proj_body(w_ref, w2_ref, o_ref):
    w2n = w2_ref[...] ** 2
    w2n = w2n / jnp.sum(w2n, axis=1, keepdims=True)
    o_ref[...] = jnp.dot(w_ref[...], w2n.T, preferred_element_type=jnp.float32)


def kernel(x, adj_mtr, w1, w2):
    del adj_mtr
    B, N, D, C = x.shape
    k = w1.shape[1]
    out_ch = w2.shape[0]
    f = D * C
    fp = (f + 127) // 128 * 128
    xf = x.reshape(B, N, f)

    nchunks = 4
    bc = B // nchunks
    outs = []
    for ci in range(nchunks):
        xc = lax.slice_in_dim(xf, ci * bc, (ci + 1) * bc, axis=0)
        idx, w1e = pl.pallas_call(
            functools.partial(_knn_body, n=N, k=k, f=f),
            grid=(bc,),
            in_specs=[
                pl.BlockSpec((1, N, f), lambda b: (b, 0, 0)),
                pl.BlockSpec((C, k), lambda b: (0, 0)),
            ],
            out_specs=[
                pl.BlockSpec((1, N, k), lambda b: (b, 0, 0)),
                pl.BlockSpec((k, f), lambda b: (0, 0)),
            ],
            out_shape=[
                jax.ShapeDtypeStruct((bc, N, k), jnp.int32),
                jax.ShapeDtypeStruct((k, f), jnp.float32),
            ],
        )(xc, w1)

        table = jnp.pad(xc.reshape(bc * N, f), ((0, 0), (0, fp - f)))
        weighted = _sc_gather_combine(table, idx.reshape(bc * N * k), w1e,
                                      k=k, f=fp,
                                      rows_per_w=bc * N // _NW)

        wflat = weighted[:, :f].reshape(bc * N * D, C)
        rows = bc * N * D
        blk = rows // 2
        out = pl.pallas_call(
            _proj_body,
            grid=(2,),
            in_specs=[
                pl.BlockSpec((blk, C), lambda i: (i, 0)),
                pl.BlockSpec((out_ch, C), lambda i: (0, 0)),
            ],
            out_specs=pl.BlockSpec((blk, out_ch), lambda i: (i, 0)),
            out_shape=jax.ShapeDtypeStruct((rows, out_ch), jnp.float32),
        )(wflat, w2)
        outs.append(out.reshape(bc, N, D, out_ch))
    return jnp.concatenate(outs, axis=0)


# speedup vs baseline: 1.0674x; 1.0674x over previous
import functools

import jax
import jax.numpy as jnp
from jax import lax
from jax.experimental import pallas as pl
from jax.experimental.pallas import tpu as pltpu
from jax.experimental.pallas import tpu_sc as plsc

_NC = 2
_NS = 16
_L = 16
_NW = _NC * _NS


def _knn_body(x_ref, w1_ref, idx_ref, w1e_ref, *, n, k, f):
    b = pl.program_id(0)
    xf = x_ref[0]
    sq = jnp.sum(xf * xf, axis=1)
    g = lax.dot_general(xf, xf, (((1,), (1,)), ((), ())),
                        preferred_element_type=jnp.float32)
    dist = sq[:, None] - 2.0 * g + sq[None, :]
    iota = lax.broadcasted_iota(jnp.int32, (n, n), 1)
    rid = lax.broadcasted_iota(jnp.int32, (n,), 0)
    ams = [rid]
    dist = jnp.where(iota == rid[:, None], jnp.inf, dist)
    for _ in range(k - 1):
        am = jnp.argmin(dist, axis=1).astype(jnp.int32)
        ams.append(am)
        dist = jnp.where(iota == am[:, None], jnp.inf, dist)
    idx = jnp.stack(ams, axis=1)
    idx_ref[0] = idx + b * n

    @pl.when(b == 0)
    def _():
        w1n = w1_ref[...] ** 2
        w1n = w1n / jnp.sum(w1n, axis=1, keepdims=True)
        w1t = w1n.T
        w1e_ref[...] = jnp.concatenate([w1t] * (f // w1t.shape[1]), axis=1)


def _sc_gather_combine(table, idxf, w1e, *, k, f, rows_per_w):
    nrows = table.shape[0]
    fc = w1e.shape[1]
    mesh = plsc.VectorSubcoreMesh(core_axis_name="c", subcore_axis_name="s")

    pt = 4
    ng = rows_per_w // pt

    @functools.partial(
        pl.kernel,
        out_type=jax.ShapeDtypeStruct((nrows, f), jnp.float32),
        mesh=mesh,
        scratch_types=[
            pltpu.VMEM((rows_per_w * k,), jnp.int32),
            pltpu.VMEM((2, pt * k, f), jnp.float32),
            pltpu.VMEM((rows_per_w, f), jnp.float32),
            pltpu.VMEM((k, fc), jnp.float32),
            pltpu.SemaphoreType.DMA,
            pltpu.SemaphoreType.DMA,
        ],
        compiler_params=pltpu.CompilerParams(use_tc_tiling_on_sc=False),
    )
    def kb(table_hbm, idx_hbm, w1e_hbm, out_hbm, idx_v, rows_v, out_v,
           w1e_v, sem0, sem1):
        wid = lax.axis_index("s") * _NC + lax.axis_index("c")
        base = wid * rows_per_w
        pltpu.sync_copy(idx_hbm.at[pl.ds(base * k, rows_per_w * k)], idx_v)
        pltpu.sync_copy(w1e_hbm, w1e_v)
        sems = (sem0, sem1)

        def gather(slot, g):
            pltpu.make_async_copy(
                table_hbm.at[idx_v.at[pl.ds(g * pt * k, pt * k)]],
                rows_v.at[slot], sems[slot],
            ).start()

        def wait(slot):
            pltpu.make_async_copy(
                table_hbm.at[idx_v.at[pl.ds(0, pt * k)]],
                rows_v.at[slot], sems[slot],
            ).wait()

        def combine(slot, g):
            for c in range(fc // _L):
                sl = pl.ds(c * _L, _L)
                acc = [None] * pt
                for j in range(k):
                    wv = w1e_v[j, sl]
                    for p in range(pt):
                        term = rows_v[slot, p * k + j, sl] * wv
                        acc[p] = term if j == 0 else acc[p] + term
                for p in range(pt):
                    out_v[g * pt + p, sl] = acc[p]

        gather(0, 0)

        def body(i, _):
            g0 = 2 * i
            gather(1, g0 + 1)
            wait(0)
            combine(0, g0)

            @pl.when(i + 1 < ng // 2)
            def _():
                gather(0, g0 + 2)

            wait(1)
            combine(1, g0 + 1)
            return 0

        lax.fori_loop(0, ng // 2, body, 0)
        pltpu.sync_copy(out_v, out_hbm.at[pl.ds(base, rows_per_w)])

    return kb(table, idxf, w1e)


def _proj_body(w_ref, w2_ref, o_ref):
    w2n = w2_ref[...] ** 2
    w2n = w2n / jnp.sum(w2n, axis=1, keepdims=True)
    o_ref[...] = jnp.dot(w_ref[...], w2n.T, preferred_element_type=jnp.float32)


def kernel(x, adj_mtr, w1, w2):
    del adj_mtr
    B, N, D, C = x.shape
    k = w1.shape[1]
    out_ch = w2.shape[0]
    f = D * C
    fp = (f + 127) // 128 * 128
    xf = x.reshape(B, N, f)

    nchunks = 2
    bc = B // nchunks
    outs = []
    for ci in range(nchunks):
        xc = lax.slice_in_dim(xf, ci * bc, (ci + 1) * bc, axis=0)
        idx, w1e = pl.pallas_call(
            functools.partial(_knn_body, n=N, k=k, f=f),
            grid=(bc,),
            in_specs=[
                pl.BlockSpec((1, N, f), lambda b: (b, 0, 0)),
                pl.BlockSpec((C, k), lambda b: (0, 0)),
            ],
            out_specs=[
                pl.BlockSpec((1, N, k), lambda b: (b, 0, 0)),
                pl.BlockSpec((k, f), lambda b: (0, 0)),
            ],
            out_shape=[
                jax.ShapeDtypeStruct((bc, N, k), jnp.int32),
                jax.ShapeDtypeStruct((k, f), jnp.float32),
            ],
        )(xc, w1)

        table = xc.reshape(bc * N, f)
        weighted = _sc_gather_combine(table, idx.reshape(bc * N * k), w1e,
                                      k=k, f=f,
                                      rows_per_w=bc * N // _NW)

        wflat = weighted.reshape(bc * N * D, C)
        rows = bc * N * D
        blk = rows // 2
        out = pl.pallas_call(
            _proj_body,
            grid=(2,),
            in_specs=[
                pl.BlockSpec((blk, C), lambda i: (i, 0)),
                pl.BlockSpec((out_ch, C), lambda i: (0, 0)),
            ],
            out_specs=pl.BlockSpec((blk, out_ch), lambda i: (i, 0)),
            out_shape=jax.ShapeDtypeStruct((rows, out_ch), jnp.float32),
        )(wflat, w2)
        outs.append(out.reshape(bc, N, D, out_ch))
    return jnp.concatenate(outs, axis=0)
